# baseline (speedup 1.0000x reference)
import jax
import jax.numpy as jnp
from jax import lax
from jax.experimental import pallas as pl
from jax.experimental.pallas import tpu as pltpu
from jax.experimental.pallas import tpu_sc as plsc

_EMB = 64
_B = 1024
_L = 50
_NW = 32
_RPW = _B // _NW
_L3, _L5, _L7 = 48, 46, 44
_NCH = _EMB // 16
_TW = 8 * _EMB
_WOFF = 7 * _EMB


def _tec_body(T_hbm, seq_hbm, o3_hbm, o5_hbm, o7_hbm,
              seq_v, t_v, o3_v, o5_v, o7_v, sem_t):
    cid = lax.axis_index("c")
    sid = lax.axis_index("s")
    wid = sid * 2 + cid
    base = wid * _RPW
    pltpu.sync_copy(seq_hbm.at[pl.ds(base, _RPW)], seq_v)

    def _row(g, carry):
        pltpu.async_copy(T_hbm.at[seq_v.at[g]], t_v, sem_t).wait()

        def _center(t, c2):
            for ch in range(_NCH):
                c0 = ch * 16
                p = [t_v[t - 3 + j, pl.ds(_WOFF + c0, 16)]
                     * t_v[t, pl.ds(j * _EMB + c0, 16)] for j in range(7)]
                m3 = jnp.maximum(jnp.maximum(p[2], p[3]), p[4])
                m5 = jnp.maximum(jnp.maximum(m3, p[1]), p[5])
                m7 = jnp.maximum(jnp.maximum(m5, p[0]), p[6])
                o3_v[t - 1, pl.ds(c0, 16)] = m3
                o5_v[t - 2, pl.ds(c0, 16)] = m5
                o7_v[t - 3, pl.ds(c0, 16)] = m7
            return c2

        lax.fori_loop(3, 47, _center, 0)

        for t, have5 in ((1, False), (2, True), (47, True), (48, False)):
            js = range(1, 6) if have5 else range(2, 5)
            for ch in range(_NCH):
                c0 = ch * 16
                p = {j: t_v[t - 3 + j, pl.ds(_WOFF + c0, 16)]
                     * t_v[t, pl.ds(j * _EMB + c0, 16)] for j in js}
                m3 = jnp.maximum(jnp.maximum(p[2], p[3]), p[4])
                o3_v[t - 1, pl.ds(c0, 16)] = m3
                if have5:
                    m5 = jnp.maximum(jnp.maximum(m3, p[1]), p[5])
                    o5_v[t - 2, pl.ds(c0, 16)] = m5

        b = base + g
        pltpu.sync_copy(o3_v, o3_hbm.at[b])
        pltpu.sync_copy(o5_v, o5_hbm.at[b])
        pltpu.sync_copy(o7_v, o7_hbm.at[b])
        return carry

    lax.fori_loop(0, _RPW, _row, 0)


def kernel(W, K, seq):
    vocab = W.shape[0]
    T = jnp.concatenate([K.reshape(vocab, _WOFF), W], axis=1)
    mesh = plsc.VectorSubcoreMesh(core_axis_name="c", subcore_axis_name="s")
    out_type = (
        jax.ShapeDtypeStruct((_B, _L3, _EMB), jnp.float32),
        jax.ShapeDtypeStruct((_B, _L5, _EMB), jnp.float32),
        jax.ShapeDtypeStruct((_B, _L7, _EMB), jnp.float32),
    )
    scratch = [
        pltpu.VMEM((_RPW, _L), jnp.int32),
        pltpu.VMEM((_L, _TW), jnp.float32),
        pltpu.VMEM((_L3, _EMB), jnp.float32),
        pltpu.VMEM((_L5, _EMB), jnp.float32),
        pltpu.VMEM((_L7, _EMB), jnp.float32),
        pltpu.SemaphoreType.DMA,
    ]
    f = pl.kernel(_tec_body, mesh=mesh, out_type=out_type,
                  scratch_types=scratch,
                  compiler_params=pltpu.CompilerParams(
                      use_tc_tiling_on_sc=False))
    return f(T, seq)


# speedup vs baseline: 2.1961x; 2.1961x over previous
import jax
import jax.numpy as jnp
from jax import lax
from jax.experimental import pallas as pl
from jax.experimental.pallas import tpu as pltpu
from jax.experimental.pallas import tpu_sc as plsc

_EMB = 64
_B = 1024
_L = 50
_NW = 32
_RPW = _B // _NW
_L3, _L5, _L7 = 48, 46, 44
_NCH = _EMB // 16


def _compute_row(w_v, k_v, o3_v, o5_v, o7_v):
    def _center(t, c2):
        for ch in range(_NCH):
            c0 = ch * 16
            p = [w_v[t - 3 + j, pl.ds(c0, 16)] * k_v[t, j, pl.ds(c0, 16)]
                 for j in range(7)]
            m3 = jnp.maximum(jnp.maximum(p[2], p[3]), p[4])
            m5 = jnp.maximum(jnp.maximum(m3, p[1]), p[5])
            m7 = jnp.maximum(jnp.maximum(m5, p[0]), p[6])
            o3_v[t - 1, pl.ds(c0, 16)] = m3
            o5_v[t - 2, pl.ds(c0, 16)] = m5
            o7_v[t - 3, pl.ds(c0, 16)] = m7
        return c2

    lax.fori_loop(3, 47, _center, 0)

    for t, have5 in ((1, False), (2, True), (47, True), (48, False)):
        js = range(1, 6) if have5 else range(2, 5)
        for ch in range(_NCH):
            c0 = ch * 16
            p = {j: w_v[t - 3 + j, pl.ds(c0, 16)] * k_v[t, j, pl.ds(c0, 16)]
                 for j in js}
            m3 = jnp.maximum(jnp.maximum(p[2], p[3]), p[4])
            o3_v[t - 1, pl.ds(c0, 16)] = m3
            if have5:
                m5 = jnp.maximum(jnp.maximum(m3, p[1]), p[5])
                o5_v[t - 2, pl.ds(c0, 16)] = m5


def _tec_body(W_hbm, K_hbm, seq_hbm, o3_hbm, o5_hbm, o7_hbm,
              seq_v, w0, w1, k0, k1,
              o30, o50, o70, o31, o51, o71,
              sg0, sg1, so0, so1):
    cid = lax.axis_index("c")
    sid = lax.axis_index("s")
    wid = sid * 2 + cid
    base = wid * _RPW
    pltpu.sync_copy(seq_hbm.at[pl.ds(base, _RPW)], seq_v)

    bufs = ((w0, k0, o30, o50, o70, sg0, so0),
            (w1, k1, o31, o51, o71, sg1, so1))

    def fire_gather(g, b):
        w_v, k_v, _, _, _, sg, _ = bufs[b]
        pltpu.async_copy(W_hbm.at[seq_v.at[g]], w_v, sg)
        pltpu.async_copy(K_hbm.at[seq_v.at[g]], k_v, sg)

    def drain_gather(b):
        w_v, k_v, _, _, _, sg, _ = bufs[b]
        pltpu.make_async_copy(W_hbm.at[seq_v.at[0]], w_v, sg).wait()
        pltpu.make_async_copy(K_hbm.at[seq_v.at[0]], k_v, sg).wait()

    def fire_out(g, b):
        _, _, o3_v, o5_v, o7_v, _, so = bufs[b]
        pltpu.async_copy(o3_v, o3_hbm.at[base + g], so)
        pltpu.async_copy(o5_v, o5_hbm.at[base + g], so)
        pltpu.async_copy(o7_v, o7_hbm.at[base + g], so)

    def drain_out(b):
        _, _, o3_v, o5_v, o7_v, _, so = bufs[b]
        pltpu.make_async_copy(o3_hbm.at[base], o3_v, so).wait()
        pltpu.make_async_copy(o5_hbm.at[base], o5_v, so).wait()
        pltpu.make_async_copy(o7_hbm.at[base], o7_v, so).wait()

    fire_gather(0, 0)
    fire_gather(1, 1)
    fire_out(0, 0)
    fire_out(1, 1)

    def _pair(i, carry):
        for b in range(2):
            g = 2 * i + b
            w_v, k_v, o3_v, o5_v, o7_v, _, _ = bufs[b]
            drain_gather(b)
            drain_out(b)
            _compute_row(w_v, k_v, o3_v, o5_v, o7_v)
            fire_gather(jnp.minimum(g + 2, _RPW - 1), b)
            fire_out(g, b)
        return carry

    lax.fori_loop(0, _RPW // 2, _pair, 0)

    for b in range(2):
        drain_gather(b)
        drain_out(b)


def kernel(W, K, seq):
    mesh = plsc.VectorSubcoreMesh(core_axis_name="c", subcore_axis_name="s")
    out_type = (
        jax.ShapeDtypeStruct((_B, _L3, _EMB), jnp.float32),
        jax.ShapeDtypeStruct((_B, _L5, _EMB), jnp.float32),
        jax.ShapeDtypeStruct((_B, _L7, _EMB), jnp.float32),
    )
    scratch = [
        pltpu.VMEM((_RPW, _L), jnp.int32),
        pltpu.VMEM((_L, _EMB), jnp.float32),
        pltpu.VMEM((_L, _EMB), jnp.float32),
        pltpu.VMEM((_L, 7, _EMB), jnp.float32),
        pltpu.VMEM((_L, 7, _EMB), jnp.float32),
        pltpu.VMEM((_L3, _EMB), jnp.float32),
        pltpu.VMEM((_L5, _EMB), jnp.float32),
        pltpu.VMEM((_L7, _EMB), jnp.float32),
        pltpu.VMEM((_L3, _EMB), jnp.float32),
        pltpu.VMEM((_L5, _EMB), jnp.float32),
        pltpu.VMEM((_L7, _EMB), jnp.float32),
        pltpu.SemaphoreType.DMA,
        pltpu.SemaphoreType.DMA,
        pltpu.SemaphoreType.DMA,
        pltpu.SemaphoreType.DMA,
    ]
    f = pl.kernel(_tec_body, mesh=mesh, out_type=out_type,
                  scratch_types=scratch,
                  compiler_params=pltpu.CompilerParams(
                      use_tc_tiling_on_sc=False))
    return f(W, K, seq)


# speedup vs baseline: 2.2053x; 1.0042x over previous
import jax
import jax.numpy as jnp
from jax import lax
from jax.experimental import pallas as pl
from jax.experimental.pallas import tpu as pltpu
from jax.experimental.pallas import tpu_sc as plsc

_EMB = 64
_B = 1024
_L = 50
_NW = 32
_RPW = _B // _NW
_L3, _L5, _L7 = 48, 46, 44
_NCH = _EMB // 16


def _compute_row(w_v, k_v, o3_v, o5_v, o7_v):
    def _center(t, c2):
        for ch in range(_NCH):
            c0 = ch * 16
            p = [w_v[t - 3 + j, pl.ds(c0, 16)] * k_v[t, j, pl.ds(c0, 16)]
                 for j in range(7)]
            m3 = jnp.maximum(jnp.maximum(p[2], p[3]), p[4])
            m5 = jnp.maximum(jnp.maximum(m3, p[1]), p[5])
            m7 = jnp.maximum(jnp.maximum(m5, p[0]), p[6])
            o3_v[t - 1, pl.ds(c0, 16)] = m3
            o5_v[t - 2, pl.ds(c0, 16)] = m5
            o7_v[t - 3, pl.ds(c0, 16)] = m7
        return c2

    lax.fori_loop(3, 47, _center, 0)

    for t, have5 in ((1, False), (2, True), (47, True), (48, False)):
        js = range(1, 6) if have5 else range(2, 5)
        for ch in range(_NCH):
            c0 = ch * 16
            p = {j: w_v[t - 3 + j, pl.ds(c0, 16)] * k_v[t, j, pl.ds(c0, 16)]
                 for j in js}
            m3 = jnp.maximum(jnp.maximum(p[2], p[3]), p[4])
            o3_v[t - 1, pl.ds(c0, 16)] = m3
            if have5:
                m5 = jnp.maximum(jnp.maximum(m3, p[1]), p[5])
                o5_v[t - 2, pl.ds(c0, 16)] = m5


_GD = 4
_OD = 2


def _tec_body(W_hbm, K_hbm, seq_hbm, o3_hbm, o5_hbm, o7_hbm,
              seq_v, w0, w1, w2, w3, k0, k1, k2, k3,
              o30, o50, o70, o31, o51, o71,
              sg0, sg1, sg2, sg3, so0, so1):
    cid = lax.axis_index("c")
    sid = lax.axis_index("s")
    wid = sid * 2 + cid
    base = wid * _RPW
    pltpu.sync_copy(seq_hbm.at[pl.ds(base, _RPW)], seq_v)

    gbufs = ((w0, k0, sg0), (w1, k1, sg1), (w2, k2, sg2), (w3, k3, sg3))
    obufs = ((o30, o50, o70, so0), (o31, o51, o71, so1))

    def fire_gather(g, b):
        w_v, k_v, sg = gbufs[b]
        pltpu.async_copy(W_hbm.at[seq_v.at[g]], w_v, sg)
        pltpu.async_copy(K_hbm.at[seq_v.at[g]], k_v, sg)

    def drain_gather(b):
        w_v, k_v, sg = gbufs[b]
        pltpu.make_async_copy(W_hbm.at[seq_v.at[0]], w_v, sg).wait()
        pltpu.make_async_copy(K_hbm.at[seq_v.at[0]], k_v, sg).wait()

    def fire_out(g, b):
        o3_v, o5_v, o7_v, so = obufs[b]
        pltpu.async_copy(o3_v, o3_hbm.at[base + g], so)
        pltpu.async_copy(o5_v, o5_hbm.at[base + g], so)
        pltpu.async_copy(o7_v, o7_hbm.at[base + g], so)

    def drain_out(b):
        o3_v, o5_v, o7_v, so = obufs[b]
        pltpu.make_async_copy(o3_hbm.at[base], o3_v, so).wait()
        pltpu.make_async_copy(o5_hbm.at[base], o5_v, so).wait()
        pltpu.make_async_copy(o7_hbm.at[base], o7_v, so).wait()

    for b in range(_GD):
        fire_gather(b, b)
    for b in range(_OD):
        fire_out(b, b)

    def _group(i, carry):
        for b in range(_GD):
            g = _GD * i + b
            w_v, k_v, _ = gbufs[b]
            o3_v, o5_v, o7_v, _ = obufs[b % _OD]
            drain_gather(b)
            drain_out(b % _OD)
            _compute_row(w_v, k_v, o3_v, o5_v, o7_v)
            fire_gather(jnp.minimum(g + _GD, _RPW - 1), b)
            fire_out(g, b % _OD)
        return carry

    lax.fori_loop(0, _RPW // _GD, _group, 0)

    for b in range(_GD):
        drain_gather(b)
    for b in range(_OD):
        drain_out(b)


def kernel(W, K, seq):
    mesh = plsc.VectorSubcoreMesh(core_axis_name="c", subcore_axis_name="s")
    out_type = (
        jax.ShapeDtypeStruct((_B, _L3, _EMB), jnp.float32),
        jax.ShapeDtypeStruct((_B, _L5, _EMB), jnp.float32),
        jax.ShapeDtypeStruct((_B, _L7, _EMB), jnp.float32),
    )
    scratch = (
        [pltpu.VMEM((_RPW, _L), jnp.int32)]
        + [pltpu.VMEM((_L, _EMB), jnp.float32) for _ in range(_GD)]
        + [pltpu.VMEM((_L, 7, _EMB), jnp.float32) for _ in range(_GD)]
        + [pltpu.VMEM((_L3, _EMB), jnp.float32),
           pltpu.VMEM((_L5, _EMB), jnp.float32),
           pltpu.VMEM((_L7, _EMB), jnp.float32)] * _OD
        + [pltpu.SemaphoreType.DMA] * (_GD + _OD)
    )
    f = pl.kernel(_tec_body, mesh=mesh, out_type=out_type,
                  scratch_types=scratch,
                  compiler_params=pltpu.CompilerParams(
                      use_tc_tiling_on_sc=False))
    return f(W, K, seq)
